# baseline (speedup 1.0000x reference)
import functools

import jax
import jax.numpy as jnp
from jax import lax
from jax.experimental import pallas as pl
from jax.experimental.pallas import tpu as pltpu
from jax.experimental.pallas import tpu_sc as plsc

B = 16384
DIM = 64
MD = 32
PD = 32
LW = 128

NC = 2
NS = 16
NW = NC * NS
BPW = B // NW
CH = 64
NCH = BPW // CH


def _sc_gather_body(u_idx, i_idx, m_idx, p_idx,
                    user_emb, item_emb, emb_manu, emb_part,
                    out_u, out_i, out_m, out_p,
                    vu_idx, vi_idx, vm_idx, vp_idx,
                    ru0, ri0, rm0, rp0, ru1, ri1, rm1, rp1,
                    s0, s1, s2, s3):
    wid = lax.axis_index("c") * NS + lax.axis_index("s")
    base = wid * BPW

    pltpu.sync_copy(u_idx.at[wid], vu_idx)
    pltpu.sync_copy(i_idx.at[wid], vi_idx)
    pltpu.sync_copy(m_idx.at[wid], vm_idx)
    pltpu.sync_copy(p_idx.at[wid], vp_idx)

    bufs = ((ru0, ri0, rm0, rp0), (ru1, ri1, rm1, rp1))
    tabs = (user_emb, item_emb, emb_manu, emb_part)
    outs = (out_u, out_i, out_m, out_p)
    idxs = (vu_idx, vi_idx, vm_idx, vp_idx)
    sems = (s0, s1, s2, s3)

    def fire(j):
        bset = bufs[j % 2]
        return [pltpu.async_copy(tabs[t].at[idxs[t].at[j]], bset[t], sems[t])
                for t in range(4)]

    pending = fire(0)
    for j in range(NCH):
        nxt = fire(j + 1) if j + 1 < NCH else None
        for c in pending:
            c.wait()
        bset = bufs[j % 2]
        off = base + j * CH
        for t in range(4):
            pltpu.sync_copy(bset[t], outs[t].at[pl.ds(off, CH)])
        pending = nxt


def _make_sc_gather():
    return functools.partial(
        pl.kernel,
        mesh=plsc.VectorSubcoreMesh(core_axis_name="c", subcore_axis_name="s"),
        out_type=[
            jax.ShapeDtypeStruct((B, LW), jnp.float32),
            jax.ShapeDtypeStruct((B, LW), jnp.float32),
            jax.ShapeDtypeStruct((B, LW), jnp.float32),
            jax.ShapeDtypeStruct((B, LW), jnp.float32),
        ],
        scratch_types=(
            [pltpu.VMEM((NCH, CH), jnp.int32) for _ in range(4)]
            + [pltpu.VMEM((CH, LW), jnp.float32) for _ in range(8)]
            + [pltpu.SemaphoreType.DMA for _ in range(4)]
        ),
    )(_sc_gather_body)


def _mlp_body(year, uid, iid, mid, pid, u128, ic128, m128, p128,
              Wy1, by1, Wy2, by2, Wp, bp, Wm1, bm1, Wm2, bm2,
              Whe, bhe, Whi, bhi,
              out_e, out_i):
    f32 = jnp.float32
    relu = lambda a: jnp.maximum(a, 0.0)
    u = jnp.where((uid[...] & 1) == 0, u128[:, 0:64], u128[:, 64:128])
    ic = jnp.where((iid[...] & 1) == 0, ic128[:, 0:64], ic128[:, 64:128])

    def pick4(idx, g):
        hi = (idx & 2) != 0
        odd = (idx & 1) != 0
        a = jnp.where(hi, g[:, 64:96], g[:, 0:32])
        b = jnp.where(hi, g[:, 96:128], g[:, 32:64])
        return jnp.where(odd, b, a)

    m = pick4(mid[...], m128)
    p = pick4(pid[...], p128)
    y1 = relu(year[...] * Wy1[...] + by1[...].reshape(1, -1))
    y = relu(jnp.dot(y1, Wy2[...], preferred_element_type=f32)
             + by2[...].reshape(1, -1))
    cin = jnp.concatenate([y, m, p], axis=1)
    cont = relu(jnp.dot(cin, Wp[...], preferred_element_type=f32)
                + bp[...].reshape(1, -1))
    x = jnp.concatenate([u, ic, cont], axis=1)
    h1 = relu(jnp.dot(x, Wm1[...], preferred_element_type=f32)
              + bm1[...].reshape(1, -1))
    h = relu(jnp.dot(h1, Wm2[...], preferred_element_type=f32)
             + bm2[...].reshape(1, -1))
    out_e[...] = jnp.dot(h, Whe[...], preferred_element_type=f32) + bhe[...]
    out_i[...] = jnp.dot(h, Whi[...], preferred_element_type=f32) + bhi[...]


def kernel(users, items, item_year, item_manu, item_part,
           user_emb, item_emb, emb_manu, emb_part,
           W_y1, b_y1, W_y2, b_y2, W_proj, b_proj,
           W_m1, b_m1, W_m2, b_m2, W_he, b_he, W_hi, b_hi, W_g, b_g):
    i32 = jnp.int32
    users = users.astype(i32)
    items = items.astype(i32)
    item_manu = item_manu.astype(i32)
    item_part = item_part.astype(i32)
    u_idx = (users >> 1).reshape(NW, NCH, CH)
    i_idx = (items >> 1).reshape(NW, NCH, CH)
    m_idx = (item_manu >> 2).reshape(NW, NCH, CH)
    p_idx = (item_part >> 2).reshape(NW, NCH, CH)

    uep = user_emb.reshape(-1, LW)
    iep = item_emb.reshape(-1, LW)
    mep = emb_manu.reshape(-1, LW)
    pep = emb_part.reshape(-1, LW)

    u_g, ic_g, m_g, p_g = _make_sc_gather()(
        u_idx, i_idx, m_idx, p_idx, uep, iep, mep, pep)

    bs = 2048
    grid = (B // bs,)
    row_spec = lambda d: pl.BlockSpec((bs, d), lambda gi: (gi, 0))
    full = lambda a: pl.BlockSpec(a.shape, lambda gi: (0,) * a.ndim)

    out_e, out_i = pl.pallas_call(
        _mlp_body,
        grid=grid,
        in_specs=[
            row_spec(1), row_spec(1), row_spec(1), row_spec(1), row_spec(1),
            row_spec(LW), row_spec(LW), row_spec(LW), row_spec(LW),
            full(W_y1), full(b_y1), full(W_y2), full(b_y2),
            full(W_proj), full(b_proj), full(W_m1), full(b_m1),
            full(W_m2), full(b_m2),
            full(W_he), full(b_he), full(W_hi), full(b_hi),
        ],
        out_specs=[pl.BlockSpec((bs, 1), lambda gi: (gi, 0)),
                   pl.BlockSpec((bs, 1), lambda gi: (gi, 0))],
        out_shape=[jax.ShapeDtypeStruct((B, 1), jnp.float32),
                   jax.ShapeDtypeStruct((B, 1), jnp.float32)],
    )(item_year, users.reshape(B, 1), items.reshape(B, 1),
      item_manu.reshape(B, 1), item_part.reshape(B, 1),
      u_g, ic_g, m_g, p_g,
      W_y1, b_y1, W_y2, b_y2, W_proj, b_proj,
      W_m1, b_m1, W_m2, b_m2, W_he, b_he, W_hi, b_hi)

    return (out_e, out_i)


# speedup vs baseline: 1.1112x; 1.1112x over previous
import functools

import jax
import jax.numpy as jnp
from jax import lax
from jax.experimental import pallas as pl
from jax.experimental.pallas import tpu as pltpu
from jax.experimental.pallas import tpu_sc as plsc

B = 16384
DIM = 64
MD = 32
PD = 32
LW = 128

NC = 2
NS = 16
NW = NC * NS
BPW = B // NW
CH = 64
NCH = BPW // CH


def _sc_gather_body(u_idx, i_idx, m_idx, p_idx,
                    user_emb, item_emb, emb_manu, emb_part,
                    out_u, out_i, out_m, out_p,
                    vu_idx, vi_idx, vm_idx, vp_idx,
                    ru0, ri0, rm0, rp0, ru1, ri1, rm1, rp1,
                    s0, s1, s2, s3):
    wid = lax.axis_index("c") * NS + lax.axis_index("s")
    base = wid * BPW

    pltpu.sync_copy(u_idx.at[wid], vu_idx)
    pltpu.sync_copy(i_idx.at[wid], vi_idx)
    pltpu.sync_copy(m_idx.at[wid], vm_idx)
    pltpu.sync_copy(p_idx.at[wid], vp_idx)

    bufs = ((ru0, ri0, rm0, rp0), (ru1, ri1, rm1, rp1))
    tabs = (user_emb, item_emb, emb_manu, emb_part)
    outs = (out_u, out_i, out_m, out_p)
    idxs = (vu_idx, vi_idx, vm_idx, vp_idx)
    sems = (s0, s1, s2, s3)

    def fire(j):
        bset = bufs[j % 2]
        return [pltpu.async_copy(tabs[t].at[idxs[t].at[j]], bset[t], sems[t])
                for t in range(4)]

    pending = fire(0)
    for j in range(NCH):
        nxt = fire(j + 1) if j + 1 < NCH else None
        for c in pending:
            c.wait()
        bset = bufs[j % 2]
        off = base + j * CH
        for t in range(4):
            pltpu.sync_copy(bset[t], outs[t].at[pl.ds(off, CH)])
        pending = nxt


def _make_sc_gather():
    return functools.partial(
        pl.kernel,
        mesh=plsc.VectorSubcoreMesh(core_axis_name="c", subcore_axis_name="s"),
        out_type=[
            jax.ShapeDtypeStruct((B, LW), jnp.float32),
            jax.ShapeDtypeStruct((B, LW), jnp.float32),
            jax.ShapeDtypeStruct((B, LW), jnp.float32),
            jax.ShapeDtypeStruct((B, LW), jnp.float32),
        ],
        scratch_types=(
            [pltpu.VMEM((NCH, CH), jnp.int32) for _ in range(4)]
            + [pltpu.VMEM((CH, LW), jnp.float32) for _ in range(8)]
            + [pltpu.SemaphoreType.DMA for _ in range(4)]
        ),
    )(_sc_gather_body)


def _mlp_body(year, mid, pid, u128, ic128, m128, p128,
              Wy1, by1, Wy2, by2, Wp, bp, Wm1, bm1, Wm2, bm2,
              Whe, bhe, Whi, bhi,
              out_e, out_i):
    f32 = jnp.float32
    relu = lambda a: jnp.maximum(a, 0.0)
    u = u128[:, 0:DIM]
    ic = ic128[:, 0:DIM]

    def pick4(idx, g):
        hi = (idx & 2) != 0
        odd = (idx & 1) != 0
        a = jnp.where(hi, g[:, 64:96], g[:, 0:32])
        b = jnp.where(hi, g[:, 96:128], g[:, 32:64])
        return jnp.where(odd, b, a)

    m = pick4(mid[...], m128)
    p = pick4(pid[...], p128)
    y1 = relu(year[...] * Wy1[...] + by1[...].reshape(1, -1))
    y = relu(jnp.dot(y1, Wy2[...], preferred_element_type=f32)
             + by2[...].reshape(1, -1))
    cin = jnp.concatenate([y, m, p], axis=1)
    cont = relu(jnp.dot(cin, Wp[...], preferred_element_type=f32)
                + bp[...].reshape(1, -1))
    x = jnp.concatenate([u, ic, cont], axis=1)
    h1 = relu(jnp.dot(x, Wm1[...], preferred_element_type=f32)
              + bm1[...].reshape(1, -1))
    h = relu(jnp.dot(h1, Wm2[...], preferred_element_type=f32)
             + bm2[...].reshape(1, -1))
    out_e[...] = jnp.dot(h, Whe[...], preferred_element_type=f32) + bhe[...]
    out_i[...] = jnp.dot(h, Whi[...], preferred_element_type=f32) + bhi[...]


def kernel(users, items, item_year, item_manu, item_part,
           user_emb, item_emb, emb_manu, emb_part,
           W_y1, b_y1, W_y2, b_y2, W_proj, b_proj,
           W_m1, b_m1, W_m2, b_m2, W_he, b_he, W_hi, b_hi, W_g, b_g):
    i32 = jnp.int32
    users = users.astype(i32)
    items = items.astype(i32)
    item_manu = item_manu.astype(i32)
    item_part = item_part.astype(i32)
    u_idx = users.reshape(NW, NCH, CH)
    i_idx = items.reshape(NW, NCH, CH)
    m_idx = (item_manu >> 2).reshape(NW, NCH, CH)
    p_idx = (item_part >> 2).reshape(NW, NCH, CH)

    uep = jnp.pad(user_emb, ((0, 0), (0, LW - DIM)))
    iep = jnp.pad(item_emb, ((0, 0), (0, LW - DIM)))
    mep = emb_manu.reshape(-1, LW)
    pep = emb_part.reshape(-1, LW)

    u_g, ic_g, m_g, p_g = _make_sc_gather()(
        u_idx, i_idx, m_idx, p_idx, uep, iep, mep, pep)

    bs = 4096
    grid = (B // bs,)
    row_spec = lambda d: pl.BlockSpec((bs, d), lambda gi: (gi, 0))
    full = lambda a: pl.BlockSpec(a.shape, lambda gi: (0,) * a.ndim)

    out_e, out_i = pl.pallas_call(
        _mlp_body,
        grid=grid,
        in_specs=[
            row_spec(1), row_spec(1), row_spec(1),
            row_spec(LW), row_spec(LW), row_spec(LW), row_spec(LW),
            full(W_y1), full(b_y1), full(W_y2), full(b_y2),
            full(W_proj), full(b_proj), full(W_m1), full(b_m1),
            full(W_m2), full(b_m2),
            full(W_he), full(b_he), full(W_hi), full(b_hi),
        ],
        out_specs=[pl.BlockSpec((bs, 1), lambda gi: (gi, 0)),
                   pl.BlockSpec((bs, 1), lambda gi: (gi, 0))],
        out_shape=[jax.ShapeDtypeStruct((B, 1), jnp.float32),
                   jax.ShapeDtypeStruct((B, 1), jnp.float32)],
    )(item_year, item_manu.reshape(B, 1), item_part.reshape(B, 1),
      u_g, ic_g, m_g, p_g,
      W_y1, b_y1, W_y2, b_y2, W_proj, b_proj,
      W_m1, b_m1, W_m2, b_m2, W_he, b_he, W_hi, b_hi)

    return (out_e, out_i)


# speedup vs baseline: 1.2095x; 1.0884x over previous
import functools

import jax
import jax.numpy as jnp
from jax import lax
from jax.experimental import pallas as pl
from jax.experimental.pallas import tpu as pltpu
from jax.experimental.pallas import tpu_sc as plsc

B = 16384
DIM = 64
MD = 32
PD = 32
LW = 128

NC = 2
NS = 16
NW = NC * NS
BPW = B // NW
CH = 64
NCH = BPW // CH


def _sc_gather_body(u_idx, i_idx, m_idx, p_idx,
                    user_emb, item_emb, emb_manu, emb_part,
                    out_u, out_i, out_m, out_p,
                    vu_idx, vi_idx, vm_idx, vp_idx,
                    ru0, ri0, rm0, rp0, ru1, ri1, rm1, rp1,
                    s0, s1, s2, s3):
    wid = lax.axis_index("c") * NS + lax.axis_index("s")
    base = wid * BPW

    pltpu.sync_copy(u_idx.at[wid], vu_idx)
    pltpu.sync_copy(i_idx.at[wid], vi_idx)
    pltpu.sync_copy(m_idx.at[wid], vm_idx)
    pltpu.sync_copy(p_idx.at[wid], vp_idx)

    bufs = ((ru0, ri0, rm0, rp0), (ru1, ri1, rm1, rp1))
    tabs = (user_emb, item_emb, emb_manu, emb_part)
    outs = (out_u, out_i, out_m, out_p)
    idxs = (vu_idx, vi_idx, vm_idx, vp_idx)
    sems = (s0, s1, s2, s3)

    def fire(j):
        bset = bufs[j % 2]
        return [pltpu.async_copy(tabs[t].at[idxs[t].at[j]], bset[t], sems[t])
                for t in range(4)]

    pending = fire(0)
    for j in range(NCH):
        nxt = fire(j + 1) if j + 1 < NCH else None
        for c in pending:
            c.wait()
        bset = bufs[j % 2]
        off = base + j * CH
        for t in range(4):
            pltpu.sync_copy(bset[t], outs[t].at[pl.ds(off, CH)])
        pending = nxt


def _make_sc_gather():
    return functools.partial(
        pl.kernel,
        mesh=plsc.VectorSubcoreMesh(core_axis_name="c", subcore_axis_name="s"),
        out_type=[
            jax.ShapeDtypeStruct((B, LW), jnp.float32),
            jax.ShapeDtypeStruct((B, LW), jnp.float32),
            jax.ShapeDtypeStruct((B, LW), jnp.float32),
            jax.ShapeDtypeStruct((B, LW), jnp.float32),
        ],
        scratch_types=(
            [pltpu.VMEM((NCH, CH), jnp.int32) for _ in range(4)]
            + [pltpu.VMEM((CH, LW), jnp.float32) for _ in range(8)]
            + [pltpu.SemaphoreType.DMA for _ in range(4)]
        ),
    )(_sc_gather_body)


def _mlp_body(year, u128, ic128, m128, p128,
              Wy1, by1, Wy2, by2, Wp, bp, Wm1, bm1, Wm2, bm2,
              Whe, bhe, Whi, bhi,
              out_e, out_i):
    f32 = jnp.float32
    relu = lambda a: jnp.maximum(a, 0.0)
    u = u128[:, 0:64]
    ic = ic128[:, 64:128]
    m = m128[:, 0:32]
    p = p128[:, 32:64]
    y1 = relu(year[...] * Wy1[...] + by1[...].reshape(1, -1))
    y = relu(jnp.dot(y1, Wy2[...], preferred_element_type=f32)
             + by2[...].reshape(1, -1))
    cin = jnp.concatenate([y, m, p], axis=1)
    cont = relu(jnp.dot(cin, Wp[...], preferred_element_type=f32)
                + bp[...].reshape(1, -1))
    x = jnp.concatenate([u, ic, cont], axis=1)
    h1 = relu(jnp.dot(x, Wm1[...], preferred_element_type=f32)
              + bm1[...].reshape(1, -1))
    h = relu(jnp.dot(h1, Wm2[...], preferred_element_type=f32)
             + bm2[...].reshape(1, -1))
    out_e[...] = jnp.dot(h, Whe[...], preferred_element_type=f32) + bhe[...]
    out_i[...] = jnp.dot(h, Whi[...], preferred_element_type=f32) + bhi[...]


def kernel(users, items, item_year, item_manu, item_part,
           user_emb, item_emb, emb_manu, emb_part,
           W_y1, b_y1, W_y2, b_y2, W_proj, b_proj,
           W_m1, b_m1, W_m2, b_m2, W_he, b_he, W_hi, b_hi, W_g, b_g):
    i32 = jnp.int32
    u_idx = users.astype(i32).reshape(NW, NCH, CH)
    i_idx = items.astype(i32).reshape(NW, NCH, CH)
    m_idx = item_manu.astype(i32).reshape(NW, NCH, CH)
    p_idx = item_part.astype(i32).reshape(NW, NCH, CH)

    ui_pack = jnp.concatenate([user_emb, item_emb], axis=1)
    mp_pack = jnp.pad(jnp.concatenate([emb_manu, emb_part], axis=1),
                      ((0, 0), (0, LW - MD - PD)))

    u_g, ic_g, m_g, p_g = _make_sc_gather()(
        u_idx, i_idx, m_idx, p_idx, ui_pack, ui_pack, mp_pack, mp_pack)

    bs = 4096
    grid = (B // bs,)
    row_spec = lambda d: pl.BlockSpec((bs, d), lambda gi: (gi, 0))
    full = lambda a: pl.BlockSpec(a.shape, lambda gi: (0,) * a.ndim)

    out_e, out_i = pl.pallas_call(
        _mlp_body,
        grid=grid,
        in_specs=[
            row_spec(1), row_spec(LW), row_spec(LW), row_spec(LW), row_spec(LW),
            full(W_y1), full(b_y1), full(W_y2), full(b_y2),
            full(W_proj), full(b_proj), full(W_m1), full(b_m1),
            full(W_m2), full(b_m2),
            full(W_he), full(b_he), full(W_hi), full(b_hi),
        ],
        out_specs=[pl.BlockSpec((bs, 1), lambda gi: (gi, 0)),
                   pl.BlockSpec((bs, 1), lambda gi: (gi, 0))],
        out_shape=[jax.ShapeDtypeStruct((B, 1), jnp.float32),
                   jax.ShapeDtypeStruct((B, 1), jnp.float32)],
    )(item_year, u_g, ic_g, m_g, p_g,
      W_y1, b_y1, W_y2, b_y2, W_proj, b_proj,
      W_m1, b_m1, W_m2, b_m2, W_he, b_he, W_hi, b_hi)

    return (out_e, out_i)


# speedup vs baseline: 1.2138x; 1.0036x over previous
import functools

import jax
import jax.numpy as jnp
from jax import lax
from jax.experimental import pallas as pl
from jax.experimental.pallas import tpu as pltpu
from jax.experimental.pallas import tpu_sc as plsc

B = 16384
DIM = 64
MD = 32
PD = 32
LW = 128

NC = 2
NS = 16
NW = NC * NS
BPW = B // NW
CH = 64
NCH = BPW // CH


def _sc_gather_body(all_idx,
                    user_emb, item_emb, emb_manu, emb_part,
                    out_u, out_i, out_m, out_p,
                    vu_idx, vi_idx, vm_idx, vp_idx,
                    ru0, ri0, rm0, rp0, ru1, ri1, rm1, rp1,
                    s0, s1, s2, s3):
    wid = lax.axis_index("c") * NS + lax.axis_index("s")
    base = wid * BPW

    pltpu.sync_copy(all_idx.at[0, wid], vu_idx)
    pltpu.sync_copy(all_idx.at[1, wid], vi_idx)
    pltpu.sync_copy(all_idx.at[2, wid], vm_idx)
    pltpu.sync_copy(all_idx.at[3, wid], vp_idx)

    bufs = ((ru0, ri0, rm0, rp0), (ru1, ri1, rm1, rp1))
    tabs = (user_emb, item_emb, emb_manu, emb_part)
    outs = (out_u, out_i, out_m, out_p)
    idxs = (vu_idx, vi_idx, vm_idx, vp_idx)
    sems = (s0, s1, s2, s3)

    def fire(j):
        bset = bufs[j % 2]
        return [pltpu.async_copy(tabs[t].at[idxs[t].at[j]], bset[t], sems[t])
                for t in range(4)]

    pending = fire(0)
    for j in range(NCH):
        nxt = fire(j + 1) if j + 1 < NCH else None
        for c in pending:
            c.wait()
        bset = bufs[j % 2]
        off = base + j * CH
        for t in range(4):
            pltpu.sync_copy(bset[t], outs[t].at[pl.ds(off, CH)])
        pending = nxt


def _make_sc_gather():
    return functools.partial(
        pl.kernel,
        mesh=plsc.VectorSubcoreMesh(core_axis_name="c", subcore_axis_name="s"),
        out_type=[
            jax.ShapeDtypeStruct((B, LW), jnp.float32),
            jax.ShapeDtypeStruct((B, LW), jnp.float32),
            jax.ShapeDtypeStruct((B, LW), jnp.float32),
            jax.ShapeDtypeStruct((B, LW), jnp.float32),
        ],
        scratch_types=(
            [pltpu.VMEM((NCH, CH), jnp.int32) for _ in range(4)]
            + [pltpu.VMEM((CH, LW), jnp.float32) for _ in range(8)]
            + [pltpu.SemaphoreType.DMA for _ in range(4)]
        ),
    )(_sc_gather_body)


def _mlp_body(year, u128, ic128, m128, p128,
              Wy1, by1, Wy2, by2, Wp, bp, Wm1, bm1, Wm2, bm2,
              Whe, bhe, Whi, bhi,
              out_e, out_i):
    f32 = jnp.float32
    relu = lambda a: jnp.maximum(a, 0.0)
    u = u128[:, 0:64]
    ic = ic128[:, 64:128]
    m = m128[:, 0:32]
    p = p128[:, 32:64]
    y1 = relu(year[...] * Wy1[...] + by1[...].reshape(1, -1))
    y = relu(jnp.dot(y1, Wy2[...], preferred_element_type=f32)
             + by2[...].reshape(1, -1))
    cin = jnp.concatenate([y, m, p], axis=1)
    cont = relu(jnp.dot(cin, Wp[...], preferred_element_type=f32)
                + bp[...].reshape(1, -1))
    x = jnp.concatenate([u, ic, cont], axis=1)
    h1 = relu(jnp.dot(x, Wm1[...], preferred_element_type=f32)
              + bm1[...].reshape(1, -1))
    h = relu(jnp.dot(h1, Wm2[...], preferred_element_type=f32)
             + bm2[...].reshape(1, -1))
    out_e[...] = jnp.dot(h, Whe[...], preferred_element_type=f32) + bhe[...]
    out_i[...] = jnp.dot(h, Whi[...], preferred_element_type=f32) + bhi[...]


def kernel(users, items, item_year, item_manu, item_part,
           user_emb, item_emb, emb_manu, emb_part,
           W_y1, b_y1, W_y2, b_y2, W_proj, b_proj,
           W_m1, b_m1, W_m2, b_m2, W_he, b_he, W_hi, b_hi, W_g, b_g):
    i32 = jnp.int32
    all_idx = jnp.stack([users.astype(i32), items.astype(i32),
                         item_manu.astype(i32), item_part.astype(i32)]
                        ).reshape(4, NW, NCH, CH)

    ui_pack = jnp.concatenate([user_emb, item_emb], axis=1)
    mp_pack = jnp.pad(jnp.concatenate([emb_manu, emb_part], axis=1),
                      ((0, 0), (0, LW - MD - PD)))

    u_g, ic_g, m_g, p_g = _make_sc_gather()(
        all_idx, ui_pack, ui_pack, mp_pack, mp_pack)

    bs = 4096
    grid = (B // bs,)
    row_spec = lambda d: pl.BlockSpec((bs, d), lambda gi: (gi, 0))
    full = lambda a: pl.BlockSpec(a.shape, lambda gi: (0,) * a.ndim)

    out_e, out_i = pl.pallas_call(
        _mlp_body,
        grid=grid,
        in_specs=[
            row_spec(1), row_spec(LW), row_spec(LW), row_spec(LW), row_spec(LW),
            full(W_y1), full(b_y1), full(W_y2), full(b_y2),
            full(W_proj), full(b_proj), full(W_m1), full(b_m1),
            full(W_m2), full(b_m2),
            full(W_he), full(b_he), full(W_hi), full(b_hi),
        ],
        out_specs=[pl.BlockSpec((bs, 1), lambda gi: (gi, 0)),
                   pl.BlockSpec((bs, 1), lambda gi: (gi, 0))],
        out_shape=[jax.ShapeDtypeStruct((B, 1), jnp.float32),
                   jax.ShapeDtypeStruct((B, 1), jnp.float32)],
    )(item_year, u_g, ic_g, m_g, p_g,
      W_y1, b_y1, W_y2, b_y2, W_proj, b_proj,
      W_m1, b_m1, W_m2, b_m2, W_he, b_he, W_hi, b_hi)

    return (out_e, out_i)
